# Initial kernel scaffold; baseline (speedup 1.0000x reference)
import jax
import jax.numpy as jnp
from jax.experimental import pallas as pl


def kernel(input, weight_int8, scale):
    raise NotImplementedError("write your pallas kernel here")


# speedup vs baseline: 8.7329x; 8.7329x over previous
import functools

import jax
import jax.numpy as jnp
from jax import lax
from jax.experimental import pallas as pl
from jax.experimental.pallas import tpu as pltpu
from jax.experimental.pallas import tpu_sc as plsc

VOCAB = 100000
EMBED_DIM = 64
BATCH = 4096
HIST = 50
N = BATCH * HIST

NUM_CORES = 2
NUM_SUBCORES = 16
NUM_WORKERS = NUM_CORES * NUM_SUBCORES
PER_WORKER = N // NUM_WORKERS
CHUNK = 640
NUM_CHUNKS = PER_WORKER // CHUNK
UNROLL = 16
WORDS = EMBED_DIM // 4

_mesh = plsc.VectorSubcoreMesh(
    core_axis_name="c", subcore_axis_name="s",
    num_cores=NUM_CORES, num_subcores=NUM_SUBCORES)


@functools.partial(
    pl.kernel,
    out_type=jax.ShapeDtypeStruct((N * EMBED_DIM,), jnp.float32),
    mesh=_mesh,
    scratch_types=[
        pltpu.VMEM((CHUNK,), jnp.int32),
        pltpu.VMEM((CHUNK, WORDS), jnp.int32),
        pltpu.VMEM((CHUNK,), jnp.float32),
        pltpu.VMEM((CHUNK * EMBED_DIM,), jnp.float32),
        pltpu.SemaphoreType.DMA,
        pltpu.SemaphoreType.DMA,
    ],
    compiler_params=pltpu.CompilerParams(use_tc_tiling_on_sc=False),
)
def _sc_embed(w_hbm, s_hbm, idx_hbm, out_hbm, idx_v, rows_v, scale_v,
              out_v, sem_w, sem_s):
    wid = lax.axis_index("s") * NUM_CORES + lax.axis_index("c")
    base_w = wid * PER_WORKER

    def chunk_body(c, _):
        base = base_w + c * CHUNK
        pltpu.sync_copy(idx_hbm.at[pl.ds(base, CHUNK)], idx_v)
        cp_w = pltpu.async_copy(w_hbm.at[idx_v], rows_v, sem_w)
        cp_s = pltpu.async_copy(s_hbm.at[idx_v], scale_v, sem_s)
        cp_w.wait()
        cp_s.wait()

        def row_body(i, _):
            sblk = scale_v[pl.ds(i * UNROLL, UNROLL)]
            for u in range(UNROLL):
                r = i * UNROLL + u
                w32 = rows_v[r]
                sv = lax.broadcast(sblk[u], (16,))
                for k in range(4):
                    x = (w32 << (24 - 8 * k)) >> 24
                    y = x.astype(jnp.float32) * sv
                    out_v[pl.ds(r * EMBED_DIM + k * 16, 16)] = y
            return ()

        lax.fori_loop(0, CHUNK // UNROLL, row_body, (), unroll=False)
        pltpu.sync_copy(
            out_v, out_hbm.at[pl.ds(base * EMBED_DIM, CHUNK * EMBED_DIM)])
        return ()

    lax.fori_loop(0, NUM_CHUNKS, chunk_body, (), unroll=False)


def kernel(input, weight_int8, scale):
    idx = input.reshape(-1).astype(jnp.int32)
    wp = weight_int8.reshape(VOCAB, 4, 16).transpose(0, 2, 1)
    w32 = lax.bitcast_convert_type(wp, jnp.int32)
    out = _sc_embed(w32, scale.reshape(-1), idx)
    return out.reshape(BATCH, HIST, EMBED_DIM)
